# baseline (speedup 1.0000x reference)
import functools

import jax
import jax.numpy as jnp
from jax import lax
from jax.experimental import pallas as pl
from jax.experimental.pallas import tpu as pltpu
from jax.experimental.pallas import tpu_sc as plsc

_LANES = 16
_NC = 2
_NS = 16
_NW = _NC * _NS
_CHUNK = 64


@functools.lru_cache(maxsize=None)
def _build(n_rows, d_model, seq_len):
    assert n_rows % _NW == 0
    per_w = n_rows // _NW
    assert per_w % _CHUNK == 0
    nch = per_w // _CHUNK
    assert seq_len % per_w == 0
    assert d_model % _LANES == 0

    mesh = plsc.VectorSubcoreMesh(core_axis_name="c", subcore_axis_name="s")

    @functools.partial(
        pl.kernel,
        mesh=mesh,
        out_type=jax.ShapeDtypeStruct((n_rows, d_model), jnp.float32),
        scratch_types=[
            pltpu.VMEM((nch, _CHUNK), jnp.int32),
            pltpu.VMEM((_CHUNK, d_model), jnp.float32),
            pltpu.VMEM((_CHUNK, d_model), jnp.float32),
            pltpu.SemaphoreType.DMA,
        ],
    )
    def emb_kernel(labels_hbm, wte_hbm, wpe_hbm, out_hbm, idx_v, rows_v, pos_v, sem):
        wid = lax.axis_index("s") * _NC + lax.axis_index("c")
        base = wid * per_w
        pos0 = lax.rem(base, seq_len)
        pltpu.sync_copy(labels_hbm.at[wid], idx_v)
        for j in range(nch):
            pltpu.sync_copy(wpe_hbm.at[pl.ds(pos0 + j * _CHUNK, _CHUNK)], pos_v)
            pltpu.async_copy(wte_hbm.at[idx_v.at[j]], rows_v, sem).wait()

            def row_body(i, _):
                def grp_body(k, _):
                    sl = pl.ds(k * _LANES, _LANES)
                    rows_v[i, sl] = rows_v[i, sl] + pos_v[i, sl]
                    return 0

                return lax.fori_loop(0, d_model // _LANES, grp_body, 0)

            lax.fori_loop(0, _CHUNK, row_body, 0)
            pltpu.sync_copy(rows_v, out_hbm.at[pl.ds(base + j * _CHUNK, _CHUNK)])

    return emb_kernel


def kernel(hidden, labels, wte, wpe):
    bsz, seq_len = labels.shape
    d_model = wte.shape[1]
    n_rows = bsz * seq_len
    per_w = n_rows // _NW
    lab = labels.astype(jnp.int32).reshape(_NW, per_w // _CHUNK, _CHUNK)
    emb = _build(n_rows, d_model, seq_len)(lab, wte, wpe)
    return (hidden, emb.reshape(bsz, seq_len, d_model), labels)


# speedup vs baseline: 1.4941x; 1.4941x over previous
import functools

import jax
import jax.numpy as jnp
from jax import lax
from jax.experimental import pallas as pl
from jax.experimental.pallas import tpu as pltpu
from jax.experimental.pallas import tpu_sc as plsc

_LANES = 16
_NC = 2
_NS = 16
_NW = _NC * _NS
_CHUNK = 32
_NBUF = 2


@functools.lru_cache(maxsize=None)
def _build(n_rows, d_model, seq_len):
    assert n_rows % _NW == 0
    per_w = n_rows // _NW
    assert per_w % _CHUNK == 0
    nch = per_w // _CHUNK
    assert seq_len % per_w == 0
    assert d_model % _LANES == 0

    mesh = plsc.VectorSubcoreMesh(core_axis_name="c", subcore_axis_name="s")

    @functools.partial(
        pl.kernel,
        mesh=mesh,
        out_type=jax.ShapeDtypeStruct((n_rows, d_model), jnp.float32),
        scratch_types=[
            pltpu.VMEM((nch, _CHUNK), jnp.int32),
            pltpu.VMEM((_NBUF, _CHUNK, d_model), jnp.float32),
            pltpu.VMEM((_NBUF, _CHUNK, d_model), jnp.float32),
            pltpu.SemaphoreType.DMA,
            pltpu.SemaphoreType.DMA,
            pltpu.SemaphoreType.DMA,
            pltpu.SemaphoreType.DMA,
        ],
    )
    def emb_kernel(labels_hbm, wte_hbm, wpe_hbm, out_hbm, idx_v, gbuf, pbuf,
                   gsem0, gsem1, psem0, psem1):
        gsems = (gsem0, gsem1)
        psems = (psem0, psem1)
        wid = lax.axis_index("s") * _NC + lax.axis_index("c")
        base = wid * per_w
        pos0 = lax.rem(base, seq_len)
        pltpu.sync_copy(labels_hbm.at[wid], idx_v)

        def start(j):
            s = j % _NBUF
            g = pltpu.async_copy(wte_hbm.at[idx_v.at[j]], gbuf.at[s], gsems[s])
            p = pltpu.async_copy(
                wpe_hbm.at[pl.ds(pos0 + j * _CHUNK, _CHUNK)], pbuf.at[s], psems[s])
            return g, p

        inflight = start(0)
        for j in range(nch):
            s = j % _NBUF
            g, p = inflight
            g.wait()
            p.wait()
            if j + 1 < nch:
                inflight = start(j + 1)

            def row_body(i, _):
                for k in range(d_model // _LANES):
                    sl = pl.ds(k * _LANES, _LANES)
                    plsc.addupdate(gbuf.at[s, i, sl], pbuf[s, i, sl])
                return 0

            lax.fori_loop(0, _CHUNK, row_body, 0)
            pltpu.sync_copy(gbuf.at[s], out_hbm.at[pl.ds(base + j * _CHUNK, _CHUNK)])

    return emb_kernel


def kernel(hidden, labels, wte, wpe):
    bsz, seq_len = labels.shape
    d_model = wte.shape[1]
    n_rows = bsz * seq_len
    per_w = n_rows // _NW
    lab = labels.astype(jnp.int32).reshape(_NW, per_w // _CHUNK, _CHUNK)
    emb = _build(n_rows, d_model, seq_len)(lab, wte, wpe)
    return (hidden, emb.reshape(bsz, seq_len, d_model), labels)


# speedup vs baseline: 1.6648x; 1.1142x over previous
import functools

import jax
import jax.numpy as jnp
from jax import lax
from jax.experimental import pallas as pl
from jax.experimental.pallas import tpu as pltpu
from jax.experimental.pallas import tpu_sc as plsc

_LANES = 16
_NC = 2
_NS = 16
_NW = _NC * _NS
_CHUNK = 32
_NBUF = 2


@functools.lru_cache(maxsize=None)
def _build(n_rows, d_model, seq_len):
    assert n_rows % _NW == 0
    per_w = n_rows // _NW
    assert per_w % _CHUNK == 0
    nch = per_w // _CHUNK
    assert seq_len % per_w == 0
    assert d_model % _LANES == 0

    mesh = plsc.VectorSubcoreMesh(core_axis_name="c", subcore_axis_name="s")

    @functools.partial(
        pl.kernel,
        mesh=mesh,
        out_type=jax.ShapeDtypeStruct((n_rows, d_model), jnp.float32),
        scratch_types=[
            pltpu.VMEM((nch, _CHUNK), jnp.int32),
            pltpu.VMEM((_NBUF, _CHUNK, d_model), jnp.float32),
            pltpu.VMEM((_NBUF, _CHUNK, d_model), jnp.float32),
            pltpu.SemaphoreType.DMA,
            pltpu.SemaphoreType.DMA,
            pltpu.SemaphoreType.DMA,
            pltpu.SemaphoreType.DMA,
        ],
    )
    def emb_kernel(labels_hbm, wte_hbm, wpe_hbm, out_hbm, idx_v, gbuf, pbuf,
                   gsem0, gsem1, psem0, psem1):
        gsems = (gsem0, gsem1)
        psems = (psem0, psem1)
        wid = lax.axis_index("s") * _NC + lax.axis_index("c")
        base = wid * per_w
        pos0 = lax.rem(base, seq_len)
        pltpu.sync_copy(labels_hbm.at[wid], idx_v)

        def start(j):
            s = j % _NBUF
            g = pltpu.async_copy(wte_hbm.at[idx_v.at[j]], gbuf.at[s], gsems[s])
            p = pltpu.async_copy(
                wpe_hbm.at[pl.ds(pos0 + j * _CHUNK, _CHUNK)], pbuf.at[s], psems[s])
            return g, p

        inflight = start(0)
        for j in range(nch):
            s = j % _NBUF
            g, p = inflight
            g.wait()
            p.wait()
            if j + 1 < nch:
                inflight = start(j + 1)

            def row_body(i, _):
                for k in range(d_model // _LANES):
                    sl = pl.ds(k * _LANES, _LANES)
                    plsc.addupdate(gbuf.at[s, i, sl], pbuf[s, i, sl])
                return 0

            lax.fori_loop(0, _CHUNK, row_body, 0)
            pltpu.sync_copy(gbuf.at[s], out_hbm.at[pl.ds(base + j * _CHUNK, _CHUNK)])

    return emb_kernel


def _copy_body(x_ref, o_ref):
    o_ref[...] = x_ref[...]


@functools.lru_cache(maxsize=None)
def _build_copy(bsz, seq_len, d_model):
    grid = (8,)
    blk = (bsz, seq_len // 8, d_model)
    return pl.pallas_call(
        _copy_body,
        grid=grid,
        in_specs=[pl.BlockSpec(blk, lambda i: (0, i, 0))],
        out_specs=pl.BlockSpec(blk, lambda i: (0, i, 0)),
        out_shape=jax.ShapeDtypeStruct((bsz, seq_len, d_model), jnp.float32),
    )


def kernel(hidden, labels, wte, wpe):
    bsz, seq_len = labels.shape
    d_model = wte.shape[1]
    n_rows = bsz * seq_len
    per_w = n_rows // _NW
    lab = labels.astype(jnp.int32).reshape(_NW, per_w // _CHUNK, _CHUNK)
    emb = _build(n_rows, d_model, seq_len)(lab, wte, wpe)
    hidden_out = _build_copy(bsz, seq_len, d_model)(hidden)
    return (hidden_out, emb.reshape(bsz, seq_len, d_model), labels)


# speedup vs baseline: 1.6905x; 1.0154x over previous
import functools

import jax
import jax.numpy as jnp
from jax import lax
from jax.experimental import pallas as pl
from jax.experimental.pallas import tpu as pltpu
from jax.experimental.pallas import tpu_sc as plsc

_LANES = 16
_NC = 2
_NS = 16
_NW = _NC * _NS
_CHUNK = 32
_NBUF = 2


@functools.lru_cache(maxsize=None)
def _build(n_rows, d_model, seq_len):
    bsz = n_rows // seq_len
    ppw = seq_len // _NW
    hb = ppw // _CHUNK
    nch = bsz * hb
    assert seq_len % (_NW * _CHUNK) == 0
    assert d_model % _LANES == 0

    mesh = plsc.VectorSubcoreMesh(core_axis_name="c", subcore_axis_name="s")

    @functools.partial(
        pl.kernel,
        mesh=mesh,
        out_type=jax.ShapeDtypeStruct((n_rows, d_model), jnp.float32),
        scratch_types=[
            pltpu.VMEM((nch, _CHUNK), jnp.int32),
            pltpu.VMEM((_NBUF, _CHUNK, d_model), jnp.float32),
            pltpu.VMEM((ppw, d_model), jnp.float32),
            pltpu.SemaphoreType.DMA,
            pltpu.SemaphoreType.DMA,
            pltpu.SemaphoreType.DMA,
            pltpu.SemaphoreType.DMA,
        ],
    )
    def emb_kernel(labels_hbm, wte_hbm, wpe_hbm, out_hbm, idx_v, gbuf, wbuf,
                   gsem0, gsem1, ssem0, ssem1):
        gsems = (gsem0, gsem1)
        ssems = (ssem0, ssem1)
        sid = lax.axis_index("s")
        cid = lax.axis_index("c")
        wid = sid * _NC + cid
        p0 = wid * ppw

        pltpu.sync_copy(wpe_hbm.at[pl.ds(p0, ppw)], wbuf)
        pltpu.sync_copy(labels_hbm.at[cid, sid], idx_v)

        def start(j):
            s = j % _NBUF
            return pltpu.async_copy(wte_hbm.at[idx_v.at[j]], gbuf.at[s], gsems[s])

        st_desc = [None] * _NBUF
        inflight = start(0)
        for j in range(nch):
            s = j % _NBUF
            nxt = (j + 1) % _NBUF
            b, h = divmod(j, hb)
            inflight.wait()
            if j + 1 < nch:
                if st_desc[nxt] is not None:
                    st_desc[nxt].wait()
                    st_desc[nxt] = None
                inflight = start(j + 1)

            def row_body(i, _, _h=h):
                for k in range(d_model // _LANES):
                    sl = pl.ds(k * _LANES, _LANES)
                    plsc.addupdate(gbuf.at[s, i, sl], wbuf[_h * _CHUNK + i, sl])
                return 0

            lax.fori_loop(0, _CHUNK, row_body, 0)
            row0 = b * seq_len + p0 + h * _CHUNK
            st_desc[s] = pltpu.async_copy(
                gbuf.at[s], out_hbm.at[pl.ds(row0, _CHUNK)], ssems[s])
        for d in st_desc:
            if d is not None:
                d.wait()

    return emb_kernel


def _copy_body(x_ref, o_ref):
    o_ref[...] = x_ref[...]


@functools.lru_cache(maxsize=None)
def _build_copy(bsz, seq_len, d_model):
    grid = (8,)
    blk = (bsz, seq_len // 8, d_model)
    return pl.pallas_call(
        _copy_body,
        grid=grid,
        in_specs=[pl.BlockSpec(blk, lambda i: (0, i, 0))],
        out_specs=pl.BlockSpec(blk, lambda i: (0, i, 0)),
        out_shape=jax.ShapeDtypeStruct((bsz, seq_len, d_model), jnp.float32),
    )


def kernel(hidden, labels, wte, wpe):
    bsz, seq_len = labels.shape
    d_model = wte.shape[1]
    n_rows = bsz * seq_len
    ppw = seq_len // _NW
    hb = ppw // _CHUNK
    lab = (labels.astype(jnp.int32)
           .reshape(bsz, _NS, _NC, hb, _CHUNK)
           .transpose(2, 1, 0, 3, 4)
           .reshape(_NC, _NS, bsz * hb, _CHUNK))
    emb = _build(n_rows, d_model, seq_len)(lab, wte, wpe)
    hidden_out = _build_copy(bsz, seq_len, d_model)(hidden)
    return (hidden_out, emb.reshape(bsz, seq_len, d_model), labels)


# speedup vs baseline: 1.9529x; 1.1552x over previous
import functools

import jax
import jax.numpy as jnp
from jax import lax
from jax.experimental import pallas as pl
from jax.experimental.pallas import tpu as pltpu
from jax.experimental.pallas import tpu_sc as plsc

_LANES = 16
_NC = 2
_NS = 16
_NW = _NC * _NS
_CHUNK = 32
_NBUF = 2


@functools.lru_cache(maxsize=None)
def _build(n_rows, d_model, seq_len):
    bsz = n_rows // seq_len
    ppw = seq_len // _NW
    hb = ppw // _CHUNK
    nch = bsz * hb
    assert seq_len % (_NW * _CHUNK) == 0
    assert d_model % _LANES == 0

    mesh = plsc.VectorSubcoreMesh(core_axis_name="c", subcore_axis_name="s")

    @functools.partial(
        pl.kernel,
        mesh=mesh,
        out_type=jax.ShapeDtypeStruct((n_rows, d_model), jnp.float32),
        scratch_types=[
            pltpu.VMEM((nch, _CHUNK), jnp.int32),
            pltpu.VMEM((_NBUF, _CHUNK, d_model), jnp.float32),
            pltpu.VMEM((ppw, d_model), jnp.float32),
            pltpu.SemaphoreType.DMA,
            pltpu.SemaphoreType.DMA,
            pltpu.SemaphoreType.DMA,
            pltpu.SemaphoreType.DMA,
        ],
    )
    def emb_kernel(labels_hbm, wte_hbm, wpe_hbm, out_hbm, idx_v, gbuf, wbuf,
                   gsem0, gsem1, ssem0, ssem1):
        gsems = (gsem0, gsem1)
        ssems = (ssem0, ssem1)
        sid = lax.axis_index("s")
        cid = lax.axis_index("c")
        wid = sid * _NC + cid
        p0 = wid * ppw

        pltpu.sync_copy(wpe_hbm.at[pl.ds(p0, ppw)], wbuf)
        pltpu.sync_copy(labels_hbm.at[cid, sid], idx_v)

        def start(j):
            s = j % _NBUF
            return pltpu.async_copy(wte_hbm.at[idx_v.at[j]], gbuf.at[s], gsems[s])

        st_desc = [None] * _NBUF
        inflight = start(0)
        for j in range(nch):
            s = j % _NBUF
            nxt = (j + 1) % _NBUF
            b, h = divmod(j, hb)
            inflight.wait()
            if j + 1 < nch:
                if st_desc[nxt] is not None:
                    st_desc[nxt].wait()
                    st_desc[nxt] = None
                inflight = start(j + 1)

            @plsc.parallel_loop(0, _CHUNK, 1, unroll=4)
            def add_rows(i, _s=s, _h=h):
                for k in range(d_model // _LANES):
                    sl = pl.ds(k * _LANES, _LANES)
                    plsc.addupdate(gbuf.at[_s, i, sl], wbuf[_h * _CHUNK + i, sl])
            row0 = b * seq_len + p0 + h * _CHUNK
            st_desc[s] = pltpu.async_copy(
                gbuf.at[s], out_hbm.at[pl.ds(row0, _CHUNK)], ssems[s])
        for d in st_desc:
            if d is not None:
                d.wait()

    return emb_kernel


def _copy_body(x_ref, o_ref):
    o_ref[...] = x_ref[...]


@functools.lru_cache(maxsize=None)
def _build_copy(bsz, seq_len, d_model):
    grid = (8,)
    blk = (bsz, seq_len // 8, d_model)
    return pl.pallas_call(
        _copy_body,
        grid=grid,
        in_specs=[pl.BlockSpec(blk, lambda i: (0, i, 0))],
        out_specs=pl.BlockSpec(blk, lambda i: (0, i, 0)),
        out_shape=jax.ShapeDtypeStruct((bsz, seq_len, d_model), jnp.float32),
    )


def kernel(hidden, labels, wte, wpe):
    bsz, seq_len = labels.shape
    d_model = wte.shape[1]
    n_rows = bsz * seq_len
    ppw = seq_len // _NW
    hb = ppw // _CHUNK
    lab = (labels.astype(jnp.int32)
           .reshape(bsz, _NS, _NC, hb, _CHUNK)
           .transpose(2, 1, 0, 3, 4)
           .reshape(_NC, _NS, bsz * hb, _CHUNK))
    emb = _build(n_rows, d_model, seq_len)(lab, wte, wpe)
    hidden_out = _build_copy(bsz, seq_len, d_model)(hidden)
    return (hidden_out, emb.reshape(bsz, seq_len, d_model), labels)
